# Initial kernel scaffold; baseline (speedup 1.0000x reference)
import jax
import jax.numpy as jnp
from jax import lax
from jax.experimental import pallas as pl
from jax.experimental.pallas import tpu as pltpu
from jax.experimental.pallas import tpu_sc as plsc

N_NODES = 100000
N_EDGES = 1600000
HIDDEN = 32
NODE_IN = 128

NC = 2
NS = 16
HALF = HIDDEN // NC
EPT = N_EDGES // NS
GROUP = 128
NFULL = EPT // GROUP
TAIL = EPT - NFULL * GROUP
DUMMY = N_NODES
ROWS_PER_TILE = N_NODES // NS
ZCHUNK = 125


def _sc_scatter_body(col_hbm, ea_hbm, agg_hbm, idx_v, rows_v, zbuf):
    c = lax.axis_index("c")
    s = lax.axis_index("s")

    def _run(agg_sh):
        def _zrow(i, _):
            zbuf[i] = jnp.zeros((HALF,), jnp.float32)
            return 0
        lax.fori_loop(0, ZCHUNK, _zrow, 0)

        def _zcopy(k, _):
            pltpu.sync_copy(zbuf,
                            agg_sh.at[pl.ds(s * ROWS_PER_TILE + k * ZCHUNK, ZCHUNK)])
            return 0
        lax.fori_loop(0, ROWS_PER_TILE // ZCHUNK, _zcopy, 0)
        plsc.subcore_barrier()

        def _group(g, _):
            e0 = s * EPT + g * GROUP
            pltpu.sync_copy(col_hbm.at[pl.ds(e0, GROUP)], idx_v)
            pltpu.sync_copy(ea_hbm.at[pl.ds(e0, GROUP), pl.ds(c * HALF, HALF)],
                            rows_v)
            pltpu.sync_copy(rows_v, agg_sh.at[idx_v], add=True)
            return 0
        lax.fori_loop(0, NFULL, _group, 0)

        for i in range(GROUP // 16):
            idx_v[pl.ds(i * 16, 16)] = jnp.full((16,), DUMMY, jnp.int32)
        e0 = s * EPT + NFULL * GROUP
        pltpu.sync_copy(col_hbm.at[pl.ds(e0, TAIL)], idx_v.at[pl.ds(0, TAIL)])
        pltpu.sync_copy(ea_hbm.at[pl.ds(e0, TAIL), pl.ds(c * HALF, HALF)],
                        rows_v.at[pl.ds(0, TAIL)])
        pltpu.sync_copy(rows_v, agg_sh.at[idx_v], add=True)

        plsc.subcore_barrier()

        r0 = s * ROWS_PER_TILE
        pltpu.sync_copy(agg_sh.at[pl.ds(r0, ROWS_PER_TILE)],
                        agg_hbm.at[pl.ds(r0, ROWS_PER_TILE), pl.ds(c * HALF, HALF)])

    pl.run_scoped(_run, pltpu.VMEM_SHARED((N_NODES + 8, HALF), jnp.float32))


def _sc_scatter(col, edge_attr):
    mesh = plsc.VectorSubcoreMesh(core_axis_name="c", subcore_axis_name="s")
    return pl.kernel(
        _sc_scatter_body,
        out_type=jax.ShapeDtypeStruct((N_NODES, HIDDEN), jnp.float32),
        mesh=mesh,
        scratch_types=[
            pltpu.VMEM((GROUP,), jnp.int32),
            pltpu.VMEM((GROUP, HALF), jnp.float32),
            pltpu.VMEM((ZCHUNK, HALF), jnp.float32),
        ],
    )(col, edge_attr)


ROW_BLOCK = 4000


def _mlp_body(x_ref, agg_ref, w1x_ref, w1a_ref, b1_ref, w2_ref, b2_ref, out_ref):
    h = jnp.dot(x_ref[...], w1x_ref[...], preferred_element_type=jnp.float32)
    h = h + jnp.dot(agg_ref[...], w1a_ref[...], preferred_element_type=jnp.float32)
    h = jnp.maximum(h + b1_ref[...], 0.0)
    h = jnp.dot(h, w2_ref[...], preferred_element_type=jnp.float32)
    out_ref[...] = jnp.maximum(h + b2_ref[...], 0.0)


def _mlp(x, agg, w1x, w1a, b1, w2, b2):
    n_blocks = N_NODES // ROW_BLOCK
    return pl.pallas_call(
        _mlp_body,
        out_shape=jax.ShapeDtypeStruct((N_NODES, HIDDEN), jnp.float32),
        grid=(n_blocks,),
        in_specs=[
            pl.BlockSpec((ROW_BLOCK, NODE_IN), lambda i: (i, 0)),
            pl.BlockSpec((ROW_BLOCK, HIDDEN), lambda i: (i, 0)),
            pl.BlockSpec((NODE_IN, HIDDEN), lambda i: (0, 0)),
            pl.BlockSpec((HIDDEN, HIDDEN), lambda i: (0, 0)),
            pl.BlockSpec((1, HIDDEN), lambda i: (0, 0)),
            pl.BlockSpec((HIDDEN, HIDDEN), lambda i: (0, 0)),
            pl.BlockSpec((1, HIDDEN), lambda i: (0, 0)),
        ],
        out_specs=pl.BlockSpec((ROW_BLOCK, HIDDEN), lambda i: (i, 0)),
    )(x, agg, w1x, w1a, b1, w2, b2)


def kernel(x, edge_index, edge_attr, u, batch, W1, b1, W2, b2):
    col = edge_index[1].astype(jnp.int32)
    agg = _sc_scatter(col, edge_attr)
    return _mlp(x, agg, W1[:NODE_IN], W1[NODE_IN:], b1.reshape(1, HIDDEN),
                W2, b2.reshape(1, HIDDEN))


# speedup vs baseline: 2.6738x; 2.6738x over previous
import jax
import jax.numpy as jnp
from jax import lax
from jax.experimental import pallas as pl
from jax.experimental.pallas import tpu as pltpu
from jax.experimental.pallas import tpu_sc as plsc

N_NODES = 100000
N_EDGES = 1600000
HIDDEN = 32
NODE_IN = 128

NC = 2
NS = 16
HALF = HIDDEN // NC
EPT = N_EDGES // NS
GROUP = 128
NFULL = EPT // GROUP
TAIL = EPT - NFULL * GROUP
DUMMY = N_NODES
ROWS_PER_TILE = N_NODES // NS
ZCHUNK = 125


def _sc_scatter_body(col_hbm, ea_hbm, agg_hbm, idx_v, rows_v, zbuf, agg_sh):
    c = lax.axis_index("c")
    s = lax.axis_index("s")

    if True:
        def _zrow(i, _):
            zbuf[i] = jnp.zeros((HALF,), jnp.float32)
            return 0
        lax.fori_loop(0, ZCHUNK, _zrow, 0)

        def _zcopy(k, _):
            pltpu.sync_copy(zbuf,
                            agg_sh.at[pl.ds(s * ROWS_PER_TILE + k * ZCHUNK, ZCHUNK)])
            return 0
        lax.fori_loop(0, ROWS_PER_TILE // ZCHUNK, _zcopy, 0)
        plsc.subcore_barrier()

        def _group(g, _):
            e0 = s * EPT + g * GROUP
            pltpu.sync_copy(col_hbm.at[pl.ds(e0, GROUP)], idx_v)
            pltpu.sync_copy(ea_hbm.at[pl.ds(e0, GROUP), pl.ds(c * HALF, HALF)],
                            rows_v)
            pltpu.sync_copy(rows_v, agg_sh.at[idx_v], add=True)
            return 0
        lax.fori_loop(0, NFULL, _group, 0)

        for i in range(GROUP // 16):
            idx_v[pl.ds(i * 16, 16)] = jnp.full((16,), DUMMY, jnp.int32)
        e0 = s * EPT + NFULL * GROUP
        pltpu.sync_copy(col_hbm.at[pl.ds(e0, TAIL)], idx_v.at[pl.ds(0, TAIL)])
        pltpu.sync_copy(ea_hbm.at[pl.ds(e0, TAIL), pl.ds(c * HALF, HALF)],
                        rows_v.at[pl.ds(0, TAIL)])
        pltpu.sync_copy(rows_v, agg_sh.at[idx_v], add=True)

        plsc.subcore_barrier()

        r0 = s * ROWS_PER_TILE
        pltpu.sync_copy(agg_sh.at[pl.ds(r0, ROWS_PER_TILE)],
                        agg_hbm.at[pl.ds(r0, ROWS_PER_TILE), pl.ds(c * HALF, HALF)])


def _sc_scatter(col, edge_attr):
    mesh = plsc.VectorSubcoreMesh(core_axis_name="c", subcore_axis_name="s")
    return pl.kernel(
        _sc_scatter_body,
        out_type=jax.ShapeDtypeStruct((N_NODES, HIDDEN), jnp.float32),
        mesh=mesh,
        scratch_types=[
            pltpu.VMEM((GROUP,), jnp.int32),
            pltpu.VMEM((GROUP, HALF), jnp.float32),
            pltpu.VMEM((ZCHUNK, HALF), jnp.float32),
            pltpu.VMEM_SHARED((N_NODES + 8, HALF), jnp.float32),
        ],
        compiler_params=pltpu.CompilerParams(use_tc_tiling_on_sc=False),
    )(col, edge_attr)


ROW_BLOCK = 4000


def _mlp_body(x_ref, agg_ref, w1x_ref, w1a_ref, b1_ref, w2_ref, b2_ref, out_ref):
    h = jnp.dot(x_ref[...], w1x_ref[...], preferred_element_type=jnp.float32)
    h = h + jnp.dot(agg_ref[...], w1a_ref[...], preferred_element_type=jnp.float32)
    h = jnp.maximum(h + b1_ref[...], 0.0)
    h = jnp.dot(h, w2_ref[...], preferred_element_type=jnp.float32)
    out_ref[...] = jnp.maximum(h + b2_ref[...], 0.0)


def _mlp(x, agg, w1x, w1a, b1, w2, b2):
    n_blocks = N_NODES // ROW_BLOCK
    return pl.pallas_call(
        _mlp_body,
        out_shape=jax.ShapeDtypeStruct((N_NODES, HIDDEN), jnp.float32),
        grid=(n_blocks,),
        in_specs=[
            pl.BlockSpec((ROW_BLOCK, NODE_IN), lambda i: (i, 0)),
            pl.BlockSpec((ROW_BLOCK, HIDDEN), lambda i: (i, 0)),
            pl.BlockSpec((NODE_IN, HIDDEN), lambda i: (0, 0)),
            pl.BlockSpec((HIDDEN, HIDDEN), lambda i: (0, 0)),
            pl.BlockSpec((1, HIDDEN), lambda i: (0, 0)),
            pl.BlockSpec((HIDDEN, HIDDEN), lambda i: (0, 0)),
            pl.BlockSpec((1, HIDDEN), lambda i: (0, 0)),
        ],
        out_specs=pl.BlockSpec((ROW_BLOCK, HIDDEN), lambda i: (i, 0)),
    )(x, agg, w1x, w1a, b1, w2, b2)


def kernel(x, edge_index, edge_attr, u, batch, W1, b1, W2, b2):
    col = edge_index[1].astype(jnp.int32)
    agg = _sc_scatter(col, edge_attr)
    return _mlp(x, agg, W1[:NODE_IN], W1[NODE_IN:], b1.reshape(1, HIDDEN),
                W2, b2.reshape(1, HIDDEN))


# speedup vs baseline: 4.5019x; 1.6837x over previous
import jax
import jax.numpy as jnp
from jax import lax
from jax.experimental import pallas as pl
from jax.experimental.pallas import tpu as pltpu
from jax.experimental.pallas import tpu_sc as plsc

N_NODES = 100000
N_EDGES = 1600000
HIDDEN = 32
NODE_IN = 128

NC = 2
NS = 16
HALF = HIDDEN // NC
GROUP = 128
NGROUPS = N_EDGES // GROUP
GPT = NGROUPS // NS
CH = 4
NCH_MAIN = (GPT // (2 * CH)) * 2
MAIN_GROUPS = NCH_MAIN * CH
STATIC_TAIL = CH
ROWS_PER_TILE = N_NODES // NS
ZCHUNK = 125


def _sc_scatter_body(col2d_hbm, ea_hbm, agg_hbm, agg_sh,
                     sem_in0, sem_in1, sem_sc0, sem_sc1):
    pl.run_scoped(
        lambda idx_v, rows_v, zbuf: _sc_scatter_inner(
            col2d_hbm, ea_hbm, agg_hbm, idx_v, rows_v, zbuf, agg_sh,
            sem_in0, sem_in1, sem_sc0, sem_sc1),
        pltpu.VMEM((2, CH, GROUP), jnp.int32),
        pltpu.VMEM((2, CH * GROUP, HALF), jnp.float32),
        pltpu.VMEM((ZCHUNK, HALF), jnp.float32),
    )


def _sc_scatter_inner(col2d_hbm, ea_hbm, agg_hbm, idx_v, rows_v, zbuf, agg_sh,
                      sem_in0, sem_in1, sem_sc0, sem_sc1):
    c = lax.axis_index("c")
    s = lax.axis_index("s")
    base = s * GPT + jnp.minimum(s, NGROUPS - GPT * NS)
    count = GPT + (s < NGROUPS - GPT * NS).astype(jnp.int32)

    def _zrow(i, _):
        zbuf[i] = jnp.zeros((HALF,), jnp.float32)
        return 0
    lax.fori_loop(0, ZCHUNK, _zrow, 0)

    def _zcopy(k, _):
        pltpu.sync_copy(zbuf,
                        agg_sh.at[pl.ds(s * ROWS_PER_TILE + k * ZCHUNK, ZCHUNK)])
        return 0
    lax.fori_loop(0, ROWS_PER_TILE // ZCHUNK, _zcopy, 0)
    plsc.subcore_barrier()

    sems_in = (sem_in0, sem_in1)
    sems_sc = (sem_sc0, sem_sc1)

    def issue_inputs(q, b):
        g0 = base + q * CH
        pltpu.async_copy(col2d_hbm.at[pl.ds(g0, CH)], idx_v.at[b], sems_in[b])
        pltpu.async_copy(
            ea_hbm.at[pl.ds(g0 * GROUP, CH * GROUP), pl.ds(c * HALF, HALF)],
            rows_v.at[b], sems_in[b])

    def wait_inputs(b):
        pltpu.make_async_copy(col2d_hbm.at[pl.ds(0, CH)], idx_v.at[b],
                              sems_in[b]).wait()
        pltpu.make_async_copy(
            ea_hbm.at[pl.ds(0, CH * GROUP), pl.ds(0, HALF)],
            rows_v.at[b], sems_in[b]).wait()

    def issue_scatters(b):
        for j in range(CH):
            pltpu.async_copy(rows_v.at[b, pl.ds(j * GROUP, GROUP)],
                             agg_sh.at[idx_v.at[b, j]], sems_sc[b], add=True)

    def wait_scatters(b):
        pltpu.make_async_copy(rows_v.at[b],
                              agg_sh.at[pl.ds(0, CH * GROUP)],
                              sems_sc[b]).wait()

    def chunk(q, b):
        o = 1 - b
        wait_inputs(b)
        issue_scatters(b)

        @pl.when(q >= 1)
        def _():
            wait_scatters(o)

        @pl.when(q + 1 < NCH_MAIN)
        def _():
            issue_inputs(q + 1, o)

    issue_inputs(0, 0)

    def _pair(p, _):
        chunk(2 * p, 0)
        chunk(2 * p + 1, 1)
        return 0
    lax.fori_loop(0, NCH_MAIN // 2, _pair, 0)
    wait_scatters(1)

    issue_inputs(NCH_MAIN, 0)
    wait_inputs(0)
    issue_scatters(0)
    wait_scatters(0)

    def _single(t, _):
        pltpu.sync_copy(col2d_hbm.at[pl.ds(base + t, 1)], idx_v.at[0, pl.ds(0, 1)])
        pltpu.sync_copy(
            ea_hbm.at[pl.ds((base + t) * GROUP, GROUP), pl.ds(c * HALF, HALF)],
            rows_v.at[0, pl.ds(0, GROUP)])
        pltpu.sync_copy(rows_v.at[0, pl.ds(0, GROUP)],
                        agg_sh.at[idx_v.at[0, 0]], add=True)
        return 0
    lax.fori_loop(MAIN_GROUPS + STATIC_TAIL, count, _single, 0)

    plsc.subcore_barrier()

    def _wb(k, _):
        r0 = s * ROWS_PER_TILE + k * ZCHUNK
        pltpu.sync_copy(agg_sh.at[pl.ds(r0, ZCHUNK)],
                        agg_hbm.at[pl.ds(r0, ZCHUNK), pl.ds(c * HALF, HALF)])
        return 0
    lax.fori_loop(0, ROWS_PER_TILE // ZCHUNK, _wb, 0)


def _sc_scatter(col2d, edge_attr):
    mesh = plsc.VectorSubcoreMesh(core_axis_name="c", subcore_axis_name="s")
    return pl.kernel(
        _sc_scatter_body,
        out_type=jax.ShapeDtypeStruct((N_NODES, HIDDEN), jnp.float32),
        mesh=mesh,
        scratch_types=[
            pltpu.VMEM_SHARED((N_NODES + 8, HALF), jnp.float32),
            pltpu.SemaphoreType.DMA,
            pltpu.SemaphoreType.DMA,
            pltpu.SemaphoreType.DMA,
            pltpu.SemaphoreType.DMA,
        ],
        compiler_params=pltpu.CompilerParams(use_tc_tiling_on_sc=False,
                                             internal_scratch_in_bytes=0),
    )(col2d, edge_attr)


ROW_BLOCK = 4000


def _mlp_body(x_ref, agg_ref, w1x_ref, w1a_ref, b1_ref, w2_ref, b2_ref, out_ref):
    h = jnp.dot(x_ref[...], w1x_ref[...], preferred_element_type=jnp.float32)
    h = h + jnp.dot(agg_ref[...], w1a_ref[...], preferred_element_type=jnp.float32)
    h = jnp.maximum(h + b1_ref[...], 0.0)
    h = jnp.dot(h, w2_ref[...], preferred_element_type=jnp.float32)
    out_ref[...] = jnp.maximum(h + b2_ref[...], 0.0)


def _mlp(x, agg, w1x, w1a, b1, w2, b2):
    n_blocks = N_NODES // ROW_BLOCK
    return pl.pallas_call(
        _mlp_body,
        out_shape=jax.ShapeDtypeStruct((N_NODES, HIDDEN), jnp.float32),
        grid=(n_blocks,),
        in_specs=[
            pl.BlockSpec((ROW_BLOCK, NODE_IN), lambda i: (i, 0)),
            pl.BlockSpec((ROW_BLOCK, HIDDEN), lambda i: (i, 0)),
            pl.BlockSpec((NODE_IN, HIDDEN), lambda i: (0, 0)),
            pl.BlockSpec((HIDDEN, HIDDEN), lambda i: (0, 0)),
            pl.BlockSpec((1, HIDDEN), lambda i: (0, 0)),
            pl.BlockSpec((HIDDEN, HIDDEN), lambda i: (0, 0)),
            pl.BlockSpec((1, HIDDEN), lambda i: (0, 0)),
        ],
        out_specs=pl.BlockSpec((ROW_BLOCK, HIDDEN), lambda i: (i, 0)),
    )(x, agg, w1x, w1a, b1, w2, b2)


def kernel(x, edge_index, edge_attr, u, batch, W1, b1, W2, b2):
    col2d = edge_index[1].astype(jnp.int32).reshape(NGROUPS, GROUP)
    agg = _sc_scatter(col2d, edge_attr)
    return _mlp(x, agg, W1[:NODE_IN], W1[NODE_IN:], b1.reshape(1, HIDDEN),
                W2, b2.reshape(1, HIDDEN))
